# baseline (speedup 1.0000x reference)
import functools

import jax
import jax.numpy as jnp
from jax import lax
from jax.experimental import pallas as pl
from jax.experimental.pallas import tpu as pltpu
from jax.experimental.pallas import tpu_sc as plsc

D = 256
K = 8192
TB = 1024

SC_NC = 2
SC_NS = 16


def _dist_argmin_body(s1_ref, s2_ref, z_ref, cb_ref, idx_ref, psum_ref, acc_ref):
    t = pl.program_id(0)
    nt = pl.num_programs(0)
    mm2 = lax.dot_general(
        z_ref[...], cb_ref[...],
        dimension_numbers=(((1,), (1,)), ((), ())),
        preferred_element_type=jnp.float32,
        precision=lax.Precision.DEFAULT,
    )
    dist = (s1_ref[...] + s2_ref[...]) - mm2
    bmin = jnp.min(dist, axis=1, keepdims=True)
    col = lax.broadcasted_iota(jnp.int32, dist.shape, 1)
    idx_ref[...] = jnp.min(
        jnp.where(dist == bmin, col, jnp.int32(K)), axis=1
    ).astype(jnp.int32).reshape(TB, 1)
    part = jnp.sum(bmin)

    @pl.when(t == 0)
    def _():
        acc_ref[0, 0] = 0.0

    acc_ref[0, 0] += part

    @pl.when(t == nt - 1)
    def _():
        psum_ref[...] = jnp.reshape(acc_ref[0, 0], (1, 1))


def _dist_argmin(s1, s2, z_flat, codebook):
    n_t = z_flat.shape[0] // TB
    return pl.pallas_call(
        _dist_argmin_body,
        grid=(n_t,),
        in_specs=[
            pl.BlockSpec((TB, 1), lambda t: (t, 0)),
            pl.BlockSpec((1, K), lambda t: (0, 0)),
            pl.BlockSpec((TB, D), lambda t: (t, 0)),
            pl.BlockSpec((K, D), lambda t: (0, 0)),
        ],
        out_specs=[
            pl.BlockSpec((TB, 1), lambda t: (t, 0)),
            pl.BlockSpec((1, 1), lambda t: (0, 0)),
        ],
        out_shape=[
            jax.ShapeDtypeStruct((z_flat.shape[0], 1), jnp.int32),
            jax.ShapeDtypeStruct((1, 1), jnp.float32),
        ],
        scratch_shapes=[pltpu.SMEM((1, 1), jnp.float32)],
    )(s1, s2, z_flat, codebook)


def _sc_gather(codebook, idx):
    n_tok = idx.shape[0]
    nw = SC_NC * SC_NS
    b_per_w = n_tok // nw
    ch = 128
    n_ch = b_per_w // ch
    mesh = plsc.VectorSubcoreMesh(core_axis_name="c", subcore_axis_name="s")

    @functools.partial(
        pl.kernel,
        mesh=mesh,
        out_type=jax.ShapeDtypeStruct((n_tok, D), jnp.float32),
        scratch_types=[
            pltpu.VMEM((n_ch, ch), jnp.int32),
            pltpu.VMEM((b_per_w, D), jnp.float32),
            pltpu.SemaphoreType.DMA,
        ],
    )
    def gather_kernel(table_hbm, idx_hbm, out_hbm, idx_v, rows_v, sem):
        wid = lax.axis_index("s") * SC_NC + lax.axis_index("c")
        base = wid * b_per_w
        for j in range(n_ch):
            pltpu.sync_copy(idx_hbm.at[pl.ds(base + j * ch, ch)], idx_v.at[j])
        dmas = [
            pltpu.async_copy(
                table_hbm.at[idx_v.at[j]], rows_v.at[pl.ds(j * ch, ch)], sem)
            for j in range(n_ch)
        ]
        for dma in dmas:
            dma.wait()
        pltpu.sync_copy(rows_v, out_hbm.at[pl.ds(base, b_per_w)])

    return gather_kernel(codebook, idx)


def kernel(z, codebook):
    b, c, h, w = z.shape
    n_tok = b * h * w
    z_ch = jnp.moveaxis(z, 1, -1)
    z_flat = z_ch.reshape(-1, D)
    s1 = jnp.sum(z_flat ** 2, axis=1, keepdims=True)
    s2 = jnp.sum(codebook ** 2, axis=1)[None, :]
    zb2 = (z_flat * 2.0).astype(jnp.bfloat16)
    cbb = codebook.astype(jnp.bfloat16)
    half = n_tok // 2
    idx_a, psum_a = _dist_argmin(s1[:half], s2, zb2[:half], cbb)
    zq_a = _sc_gather(codebook, idx_a.reshape(-1))
    idx_b, psum_b = _dist_argmin(s1[half:], s2, zb2[half:], cbb)
    zq_b = _sc_gather(codebook, idx_b.reshape(-1))
    zq_flat = jnp.concatenate([zq_a, zq_b], axis=0)
    m = (psum_a[0, 0] + psum_b[0, 0]) * (1.0 / 2097152.0)
    loss = 0.25 * m + m
    z_q = jnp.moveaxis(zq_flat.reshape(b, h, w, c), -1, 1)
    return (z_q, loss)


# speedup vs baseline: 1.1049x; 1.1049x over previous
import functools

import jax
import jax.numpy as jnp
from jax import lax
from jax.experimental import pallas as pl
from jax.experimental.pallas import tpu as pltpu
from jax.experimental.pallas import tpu_sc as plsc

D = 256
K = 8192
TB = 1024

SC_NC = 2
SC_NS = 16


def _dist_argmin_body(s1_ref, s2_ref, z_ref, cb_ref, idx_ref, psum_ref, acc_ref):
    t = pl.program_id(0)
    nt = pl.num_programs(0)
    mm2 = lax.dot_general(
        z_ref[...], cb_ref[...],
        dimension_numbers=(((1,), (1,)), ((), ())),
        preferred_element_type=jnp.float32,
        precision=lax.Precision.DEFAULT,
    )
    dist = (s1_ref[...] + s2_ref[...]) - mm2
    bmin = jnp.min(dist, axis=1, keepdims=True)
    col = lax.broadcasted_iota(jnp.int32, dist.shape, 1)
    idx_ref[...] = jnp.min(
        jnp.where(dist == bmin, col, jnp.int32(K)), axis=1
    ).astype(jnp.int32).reshape(TB, 1)
    part = jnp.sum(bmin)

    @pl.when(t == 0)
    def _():
        acc_ref[0, 0] = 0.0

    acc_ref[0, 0] += part

    @pl.when(t == nt - 1)
    def _():
        psum_ref[...] = jnp.reshape(acc_ref[0, 0], (1, 1))


def _dist_argmin(s1, s2, z_flat, codebook):
    n_t = z_flat.shape[0] // TB
    return pl.pallas_call(
        _dist_argmin_body,
        grid=(n_t,),
        in_specs=[
            pl.BlockSpec((TB, 1), lambda t: (t, 0)),
            pl.BlockSpec((1, K), lambda t: (0, 0)),
            pl.BlockSpec((TB, D), lambda t: (t, 0)),
            pl.BlockSpec((K, D), lambda t: (0, 0)),
        ],
        out_specs=[
            pl.BlockSpec((TB, 1), lambda t: (t, 0)),
            pl.BlockSpec((1, 1), lambda t: (0, 0)),
        ],
        out_shape=[
            jax.ShapeDtypeStruct((z_flat.shape[0], 1), jnp.int32),
            jax.ShapeDtypeStruct((1, 1), jnp.float32),
        ],
        scratch_shapes=[pltpu.SMEM((1, 1), jnp.float32)],
    )(s1, s2, z_flat, codebook)


def _sc_gather(codebook, idx):
    n_tok = idx.shape[0]
    nw = SC_NC * SC_NS
    b_per_w = n_tok // nw
    ch = 128
    n_ch = b_per_w // ch
    mesh = plsc.VectorSubcoreMesh(core_axis_name="c", subcore_axis_name="s")

    @functools.partial(
        pl.kernel,
        mesh=mesh,
        out_type=jax.ShapeDtypeStruct((n_tok, D), jnp.float32),
        scratch_types=[
            pltpu.VMEM((n_ch, ch), jnp.int32),
            pltpu.VMEM((b_per_w, D), jnp.float32),
            pltpu.SemaphoreType.DMA,
        ],
    )
    def gather_kernel(table_hbm, idx_hbm, out_hbm, idx_v, rows_v, sem):
        wid = lax.axis_index("s") * SC_NC + lax.axis_index("c")
        base = wid * b_per_w
        for j in range(n_ch):
            pltpu.sync_copy(idx_hbm.at[pl.ds(base + j * ch, ch)], idx_v.at[j])
        dmas = [
            pltpu.async_copy(
                table_hbm.at[idx_v.at[j]], rows_v.at[pl.ds(j * ch, ch)], sem)
            for j in range(n_ch)
        ]
        for dma in dmas:
            dma.wait()
        pltpu.sync_copy(rows_v, out_hbm.at[pl.ds(base, b_per_w)])

    return gather_kernel(codebook, idx)


def kernel(z, codebook):
    b, c, h, w = z.shape
    n_tok = b * h * w
    z_ch = jnp.moveaxis(z, 1, -1)
    z_flat = z_ch.reshape(-1, D)
    s1 = jnp.sum(z_flat ** 2, axis=1, keepdims=True)
    s2 = jnp.sum(codebook ** 2, axis=1)[None, :]
    zb2 = (z_flat * 2.0).astype(jnp.bfloat16)
    cbb = codebook.astype(jnp.bfloat16)
    idx2d, psum = _dist_argmin(s1, s2, zb2, cbb)
    zq_flat = _sc_gather(codebook, idx2d.reshape(-1))
    m = psum[0, 0] * (1.0 / float(n_tok * D))
    loss = 0.25 * m + m
    z_q = jnp.moveaxis(zq_flat.reshape(b, h, w, c), -1, 1)
    return (z_q, loss)


# speedup vs baseline: 1.1357x; 1.0279x over previous
import functools

import jax
import jax.numpy as jnp
from jax import lax
from jax.experimental import pallas as pl
from jax.experimental.pallas import tpu as pltpu
from jax.experimental.pallas import tpu_sc as plsc

D = 256
K = 8192
TB = 1024

SC_NC = 2
SC_NS = 16


def _dist_argmin_body(s1_ref, s2_ref, z_ref, cb_ref, idx_ref, psum_ref, acc_ref):
    t = pl.program_id(0)
    nt = pl.num_programs(0)
    mm2 = lax.dot_general(
        z_ref[...], cb_ref[...],
        dimension_numbers=(((1,), (1,)), ((), ())),
        preferred_element_type=jnp.float32,
        precision=lax.Precision.DEFAULT,
    )
    dist = (s1_ref[...] + s2_ref[...]) - mm2
    bmin = jnp.min(dist, axis=1, keepdims=True)
    col = lax.broadcasted_iota(jnp.int32, dist.shape, 1)
    idx_ref[...] = jnp.min(
        jnp.where(dist == bmin, col, jnp.int32(K)), axis=1
    ).astype(jnp.int32).reshape(TB, 1)
    part = jnp.sum(bmin)

    @pl.when(t == 0)
    def _():
        acc_ref[0, 0] = 0.0

    acc_ref[0, 0] += part

    @pl.when(t == nt - 1)
    def _():
        m = acc_ref[0, 0] * (1.0 / 2097152.0)
        psum_ref[...] = jnp.reshape(0.25 * m + m, (1, 1))


def _dist_argmin(s1, s2, z_flat, codebook):
    n_t = z_flat.shape[0] // TB
    return pl.pallas_call(
        _dist_argmin_body,
        grid=(n_t,),
        in_specs=[
            pl.BlockSpec((TB, 1), lambda t: (t, 0)),
            pl.BlockSpec((1, K), lambda t: (0, 0)),
            pl.BlockSpec((TB, D), lambda t: (t, 0)),
            pl.BlockSpec((K, D), lambda t: (0, 0)),
        ],
        out_specs=[
            pl.BlockSpec((TB, 1), lambda t: (t, 0)),
            pl.BlockSpec((1, 1), lambda t: (0, 0)),
        ],
        out_shape=[
            jax.ShapeDtypeStruct((z_flat.shape[0], 1), jnp.int32),
            jax.ShapeDtypeStruct((1, 1), jnp.float32),
        ],
        scratch_shapes=[pltpu.SMEM((1, 1), jnp.float32)],
    )(s1, s2, z_flat, codebook)


def _sc_gather(codebook, idx):
    n_tok = idx.shape[0]
    nw = SC_NC * SC_NS
    b_per_w = n_tok // nw
    ch = 128
    n_ch = b_per_w // ch
    mesh = plsc.VectorSubcoreMesh(core_axis_name="c", subcore_axis_name="s")

    @functools.partial(
        pl.kernel,
        mesh=mesh,
        out_type=jax.ShapeDtypeStruct((n_tok, D), jnp.float32),
        scratch_types=[
            pltpu.VMEM((n_ch, ch), jnp.int32),
            pltpu.VMEM((b_per_w, D), jnp.float32),
            pltpu.SemaphoreType.DMA,
        ],
    )
    def gather_kernel(table_hbm, idx_hbm, out_hbm, idx_v, rows_v, sem):
        wid = lax.axis_index("s") * SC_NC + lax.axis_index("c")
        base = wid * b_per_w
        for j in range(n_ch):
            pltpu.sync_copy(idx_hbm.at[pl.ds(base + j * ch, ch)], idx_v.at[j])
        dmas = [
            pltpu.async_copy(
                table_hbm.at[idx_v.at[j]], rows_v.at[pl.ds(j * ch, ch)], sem)
            for j in range(n_ch)
        ]
        for dma in dmas:
            dma.wait()
        pltpu.sync_copy(rows_v, out_hbm.at[pl.ds(base, b_per_w)])

    return gather_kernel(codebook, idx)


def kernel(z, codebook):
    b, c, h, w = z.shape
    n_tok = b * h * w
    z_ch = jnp.moveaxis(z, 1, -1)
    z_flat = z_ch.reshape(-1, D)
    s1 = jnp.sum(z_flat ** 2, axis=1, keepdims=True)
    s2 = jnp.sum(codebook ** 2, axis=1)[None, :]
    zb2 = (z_flat * 2.0).astype(jnp.bfloat16)
    cbb = codebook.astype(jnp.bfloat16)
    idx2d, loss11 = _dist_argmin(s1, s2, zb2, cbb)
    zq_flat = _sc_gather(codebook, idx2d.reshape(-1))
    loss = loss11[0, 0]
    z_q = jnp.moveaxis(zq_flat.reshape(b, h, w, c), -1, 1)
    return (z_q, loss)
